# baseline (speedup 1.0000x reference)
import jax
import jax.numpy as jnp
from jax import lax
from jax.experimental import pallas as pl
from jax.experimental.pallas import tpu as pltpu
from jax.experimental.pallas import tpu_sc as plsc

_NC, _NS, _L = 2, 16, 16
_NW = _NC * _NS
_B, _S, _D = 4096, 128, 128
_RPW = _S // _NW
_KPR = _D // _L
_NB = 32
_UNROLL = 8
_NCHUNK = _B // _NB


def _sc_body(x_hbm, t_hbm, o_hbm, tbuf, in0, in1, ou0, ou1, si0, si1, so0, so1):
    wid = lax.axis_index("s") * _NC + lax.axis_index("c")
    r0 = wid * _RPW
    ins, ous, sis, sos = (in0, in1), (ou0, ou1), (si0, si1), (so0, so1)

    pltpu.sync_copy(t_hbm.at[pl.ds(r0, _RPW), :], tbuf)
    tv = [tbuf[r, pl.ds(k * _L, _L)] for r in range(_RPW) for k in range(_KPR)]

    def start_in(c, b):
        pltpu.make_async_copy(
            x_hbm.at[pl.ds(c * _NB, _NB), pl.ds(r0, _RPW), :], ins[b], sis[b]
        ).start()

    def wait_in(b):
        pltpu.make_async_copy(
            x_hbm.at[pl.ds(0, _NB), pl.ds(r0, _RPW), :], ins[b], sis[b]
        ).wait()

    def start_out(c, b):
        pltpu.make_async_copy(
            ous[b], o_hbm.at[pl.ds(c * _NB, _NB), pl.ds(r0, _RPW), :], sos[b]
        ).start()

    def wait_out(b):
        pltpu.make_async_copy(
            ous[b], o_hbm.at[pl.ds(0, _NB), pl.ds(r0, _RPW), :], sos[b]
        ).wait()

    def compute(b):
        inb, oub = ins[b], ous[b]

        def body(it, _):
            for u in range(_UNROLL):
                bi = it * _UNROLL + u
                for r in range(_RPW):
                    for k in range(_KPR):
                        sl = pl.ds(k * _L, _L)
                        oub[bi, r, sl] = inb[bi, r, sl] + tv[r * _KPR + k]
            return ()

        lax.fori_loop(0, _NB // _UNROLL, body, ())

    start_in(0, 0)
    start_in(1, 1)
    for c in (0, 1):
        b = c
        wait_in(b)
        compute(b)
        start_out(c, b)
        start_in(c + 2, b)

    def outer(o, _):
        c0 = o * 2
        for b in range(2):
            c = c0 + b
            wait_in(b)
            wait_out(b)
            compute(b)
            start_out(c, b)
            start_in(c + 2, b)
        return ()

    lax.fori_loop(1, _NCHUNK // 2 - 1, outer, ())

    for c in (_NCHUNK - 2, _NCHUNK - 1):
        b = c % 2
        wait_in(b)
        wait_out(b)
        compute(b)
        start_out(c, b)
    wait_out(0)
    wait_out(1)


_sc_kernel = pl.kernel(
    _sc_body,
    out_type=jax.ShapeDtypeStruct((_B, _S, _D), jnp.float32),
    mesh=plsc.VectorSubcoreMesh(
        core_axis_name="c", subcore_axis_name="s", num_cores=_NC, num_subcores=_NS
    ),
    scratch_types=[
        pltpu.VMEM((_RPW, _D), jnp.float32),
        pltpu.VMEM((_NB, _RPW, _D), jnp.float32),
        pltpu.VMEM((_NB, _RPW, _D), jnp.float32),
        pltpu.VMEM((_NB, _RPW, _D), jnp.float32),
        pltpu.VMEM((_NB, _RPW, _D), jnp.float32),
        pltpu.SemaphoreType.DMA,
        pltpu.SemaphoreType.DMA,
        pltpu.SemaphoreType.DMA,
        pltpu.SemaphoreType.DMA,
    ],
)


def kernel(x, table):
    return _sc_kernel(x, table)


# speedup vs baseline: 1.1850x; 1.1850x over previous
import jax
import jax.numpy as jnp
from jax import lax
from jax.experimental import pallas as pl
from jax.experimental.pallas import tpu as pltpu
from jax.experimental.pallas import tpu_sc as plsc

_NC, _NS, _L = 2, 16, 16
_NW = _NC * _NS
_B, _S, _D = 4096, 128, 128
_RPW = _S // _NW
_KPR = _D // _L
_NB = 32
_UNROLL = 4
_NCHUNK = _B // _NB


def _sc_body(x_hbm, t_hbm, o_hbm, tbuf, in0, in1, ou0, ou1, si0, si1, so0, so1):
    wid = lax.axis_index("s") * _NC + lax.axis_index("c")
    r0 = wid * _RPW
    ins, ous, sis, sos = (in0, in1), (ou0, ou1), (si0, si1), (so0, so1)

    pltpu.sync_copy(t_hbm.at[pl.ds(r0, _RPW), :], tbuf)
    tv = [tbuf[r, pl.ds(k * _L, _L)] for r in range(_RPW) for k in range(_KPR)]

    def start_in(c, b):
        pltpu.make_async_copy(
            x_hbm.at[pl.ds(c * _NB, _NB), pl.ds(r0, _RPW), :], ins[b], sis[b]
        ).start()

    def wait_in(b):
        pltpu.make_async_copy(
            x_hbm.at[pl.ds(0, _NB), pl.ds(r0, _RPW), :], ins[b], sis[b]
        ).wait()

    def start_out(c, b):
        pltpu.make_async_copy(
            ous[b], o_hbm.at[pl.ds(c * _NB, _NB), pl.ds(r0, _RPW), :], sos[b]
        ).start()

    def wait_out(b):
        pltpu.make_async_copy(
            ous[b], o_hbm.at[pl.ds(0, _NB), pl.ds(r0, _RPW), :], sos[b]
        ).wait()

    def compute(b):
        inb, oub = ins[b], ous[b]

        def body(it, _):
            for u in range(_UNROLL):
                bi = it * _UNROLL + u
                for r in range(_RPW):
                    for k in range(_KPR):
                        sl = pl.ds(k * _L, _L)
                        oub[bi, r, sl] = inb[bi, r, sl] + tv[r * _KPR + k]
            return ()

        lax.fori_loop(0, _NB // _UNROLL, body, ())

    start_in(0, 0)
    start_in(1, 1)

    def outer(o, _):
        for b in range(2):
            c = o * 2 + b
            wait_in(b)

            @pl.when(o >= 1)
            def _():
                wait_out(b)

            compute(b)
            start_out(c, b)

            @pl.when(o < _NCHUNK // 2 - 1)
            def _():
                start_in(c + 2, b)

        return ()

    lax.fori_loop(0, _NCHUNK // 2, outer, ())
    wait_out(0)
    wait_out(1)


_sc_kernel = pl.kernel(
    _sc_body,
    out_type=jax.ShapeDtypeStruct((_B, _S, _D), jnp.float32),
    mesh=plsc.VectorSubcoreMesh(
        core_axis_name="c", subcore_axis_name="s", num_cores=_NC, num_subcores=_NS
    ),
    scratch_types=[
        pltpu.VMEM((_RPW, _D), jnp.float32),
        pltpu.VMEM((_NB, _RPW, _D), jnp.float32),
        pltpu.VMEM((_NB, _RPW, _D), jnp.float32),
        pltpu.VMEM((_NB, _RPW, _D), jnp.float32),
        pltpu.VMEM((_NB, _RPW, _D), jnp.float32),
        pltpu.SemaphoreType.DMA,
        pltpu.SemaphoreType.DMA,
        pltpu.SemaphoreType.DMA,
        pltpu.SemaphoreType.DMA,
    ],
)


def kernel(x, table):
    return _sc_kernel(x, table)
